# Initial kernel scaffold; baseline (speedup 1.0000x reference)
import jax
import jax.numpy as jnp
from jax.experimental import pallas as pl


def kernel(x, edge_index, W1l, b1l, W1r, W2l, b2l, W2r, gamma, beta):
    raise NotImplementedError("write your pallas kernel here")


# speedup vs baseline: 4.8291x; 4.8291x over previous
import jax
import jax.numpy as jnp
from jax import lax
from jax.experimental import pallas as pl
from jax.experimental.pallas import tpu as pltpu
from jax.experimental.pallas import tpu_sc as plsc

N = 10000
E = 320000
D = 128

NC = 2
NS = 16
NW = NC * NS
EPW = E // NW
K = 80
NCH = EPW // K
RPS = N // NS
ZR = 25

_MESH = plsc.VectorSubcoreMesh(core_axis_name="c", subcore_axis_name="s")


def _fill_rows(ref, nrows, value):
    def fill(i, _):
        for j in range(D // 16):
            ref[i, pl.ds(j * 16, 16)] = jnp.full((16,), value, jnp.float32)
        return 0
    lax.fori_loop(0, nrows, fill, 0)


def _zero_shared(zrows, sh, s):
    for r in range(RPS // ZR):
        base = s * RPS + r * ZR
        pltpu.sync_copy(zrows, sh.at[pl.ds(base, ZR)])


def _sc_agg_body(x_hbm, src_hbm, dst_hbm, acc_out, acc_sh, src_v, dst_v,
                 rows_v, zrows, sem):
    c = lax.axis_index("c")
    s = lax.axis_index("s")
    wid = c * NS + s

    _fill_rows(zrows, ZR, 0.0)
    _zero_shared(zrows, acc_sh, s)
    plsc.subcore_barrier()

    def chunk(i, _):
        base = wid * EPW + i * K
        pltpu.sync_copy(src_hbm.at[pl.ds(base, K)], src_v)
        pltpu.sync_copy(dst_hbm.at[pl.ds(base, K)], dst_v)
        pltpu.async_copy(x_hbm.at[src_v], rows_v, sem).wait()
        pltpu.sync_copy(rows_v, acc_sh.at[dst_v], add=True)
        return 0
    lax.fori_loop(0, NCH, chunk, 0)
    plsc.subcore_barrier()

    @pl.when(s == 0)
    def _flush():
        pltpu.sync_copy(acc_sh, acc_out.at[c])


_sc_agg = pl.kernel(
    _sc_agg_body,
    out_type=jax.ShapeDtypeStruct((NC, N, D), jnp.float32),
    mesh=_MESH,
    scratch_types=[
        pltpu.VMEM_SHARED((N, D), jnp.float32),
        pltpu.VMEM((K,), jnp.int32),
        pltpu.VMEM((K,), jnp.int32),
        pltpu.VMEM((K, D), jnp.float32),
        pltpu.VMEM((ZR, D), jnp.float32),
        pltpu.SemaphoreType.DMA,
    ],
)


def _sc_deg_body(dst_hbm, deg_out, deg_sh, dst_v, ones_v, zrows, sem):
    del sem
    c = lax.axis_index("c")
    s = lax.axis_index("s")
    wid = c * NS + s

    _fill_rows(zrows, ZR, 0.0)
    _fill_rows(ones_v, K, 1.0)
    _zero_shared(zrows, deg_sh, s)
    plsc.subcore_barrier()

    def chunk(i, _):
        base = wid * EPW + i * K
        pltpu.sync_copy(dst_hbm.at[pl.ds(base, K)], dst_v)
        pltpu.sync_copy(ones_v, deg_sh.at[dst_v], add=True)
        return 0
    lax.fori_loop(0, NCH, chunk, 0)
    plsc.subcore_barrier()

    @pl.when(s == 0)
    def _flush():
        pltpu.sync_copy(deg_sh, deg_out.at[c])


_sc_deg = pl.kernel(
    _sc_deg_body,
    out_type=jax.ShapeDtypeStruct((NC, N, D), jnp.float32),
    mesh=_MESH,
    scratch_types=[
        pltpu.VMEM_SHARED((N, D), jnp.float32),
        pltpu.VMEM((K,), jnp.int32),
        pltpu.VMEM((K, D), jnp.float32),
        pltpu.VMEM((ZR, D), jnp.float32),
        pltpu.SemaphoreType.DMA,
    ],
)


def _dense1_body(x_ref, acc_ref, deg_ref, Wl_ref, bl_ref, Wr_ref,
                 gamma_ref, beta_ref, h_ref):
    a = acc_ref[0] + acc_ref[1]
    cnt = deg_ref[0][:, 0:1] + deg_ref[1][:, 0:1]
    mean = a / jnp.maximum(cnt, 1.0)
    out = (jnp.dot(mean, Wl_ref[...], preferred_element_type=jnp.float32)
           + bl_ref[...][None, :]
           + jnp.dot(x_ref[...], Wr_ref[...], preferred_element_type=jnp.float32))
    nrm = jnp.sqrt(jnp.sum(out * out, axis=1, keepdims=True))
    out = out / jnp.maximum(nrm, 1e-12)
    h = jnp.maximum(out, 0.0)
    mu = jnp.mean(h, axis=0, keepdims=True)
    var = jnp.mean((h - mu) * (h - mu), axis=0, keepdims=True)
    h_ref[...] = (gamma_ref[...][None, :] * (h - mu) * lax.rsqrt(var + 1e-5)
                  + beta_ref[...][None, :])


def _dense2_body(h_ref, acc_ref, deg_ref, Wl_ref, bl_ref, Wr_ref, o_ref):
    a = acc_ref[0] + acc_ref[1]
    cnt = deg_ref[0][:, 0:1] + deg_ref[1][:, 0:1]
    mean = a / jnp.maximum(cnt, 1.0)
    out = (jnp.dot(mean, Wl_ref[...], preferred_element_type=jnp.float32)
           + bl_ref[...][None, :]
           + jnp.dot(h_ref[...], Wr_ref[...], preferred_element_type=jnp.float32))
    nrm = jnp.sqrt(jnp.sum(out * out, axis=1, keepdims=True))
    o_ref[...] = out / jnp.maximum(nrm, 1e-12)


_dense1 = pl.pallas_call(
    _dense1_body, out_shape=jax.ShapeDtypeStruct((N, D), jnp.float32))
_dense2 = pl.pallas_call(
    _dense2_body, out_shape=jax.ShapeDtypeStruct((N, D), jnp.float32))


@jax.jit
def kernel(x, edge_index, W1l, b1l, W1r, W2l, b2l, W2r, gamma, beta):
    src = edge_index[0]
    dst = edge_index[1]
    deg = _sc_deg(dst)
    acc1 = _sc_agg(x, src, dst)
    h = _dense1(x, acc1, deg, W1l, b1l, W1r, gamma, beta)
    acc2 = _sc_agg(h, src, dst)
    return _dense2(h, acc2, deg, W2l, b2l, W2r)


# speedup vs baseline: 6.2055x; 1.2850x over previous
import jax
import jax.numpy as jnp
from jax import lax
from jax.experimental import pallas as pl
from jax.experimental.pallas import tpu as pltpu
from jax.experimental.pallas import tpu_sc as plsc

N = 10000
E = 320000
D = 128

NC = 2
NS = 16
NW = NC * NS
EPW = E // NW
K = 80
NCH = EPW // K
RPS = N // NS
ZR = 25

_MESH = plsc.VectorSubcoreMesh(core_axis_name="c", subcore_axis_name="s")


def _fill_rows(ref, nrows, value):
    def fill(i, _):
        for j in range(D // 16):
            ref[i, pl.ds(j * 16, 16)] = jnp.full((16,), value, jnp.float32)
        return 0
    lax.fori_loop(0, nrows, fill, 0)


def _zero_shared(zrows, sh, s):
    for r in range(RPS // ZR):
        base = s * RPS + r * ZR
        pltpu.sync_copy(zrows, sh.at[pl.ds(base, ZR)])


def _sc_agg_body(x_hbm, src_hbm, dst_hbm, acc_out, acc_sh,
                 src0, dst0, rows0, sem0, src1, dst1, rows1, sem1, zrows):
    c = lax.axis_index("c")
    s = lax.axis_index("s")
    wid = c * NS + s

    _fill_rows(zrows, ZR, 0.0)
    _zero_shared(zrows, acc_sh, s)
    plsc.subcore_barrier()

    def pair(t, _):
        base0 = wid * EPW + (2 * t) * K
        base1 = base0 + K
        pltpu.sync_copy(src_hbm.at[pl.ds(base0, K)], src0)
        pltpu.sync_copy(dst_hbm.at[pl.ds(base0, K)], dst0)
        g0 = pltpu.async_copy(x_hbm.at[src0], rows0, sem0)
        pltpu.sync_copy(src_hbm.at[pl.ds(base1, K)], src1)
        pltpu.sync_copy(dst_hbm.at[pl.ds(base1, K)], dst1)
        g1 = pltpu.async_copy(x_hbm.at[src1], rows1, sem1)
        g0.wait()
        pltpu.sync_copy(rows0, acc_sh.at[dst0], add=True)
        g1.wait()
        pltpu.sync_copy(rows1, acc_sh.at[dst1], add=True)
        return 0
    lax.fori_loop(0, NCH // 2, pair, 0)
    if NCH % 2:
        base = wid * EPW + (NCH - 1) * K
        pltpu.sync_copy(src_hbm.at[pl.ds(base, K)], src0)
        pltpu.sync_copy(dst_hbm.at[pl.ds(base, K)], dst0)
        pltpu.async_copy(x_hbm.at[src0], rows0, sem0).wait()
        pltpu.sync_copy(rows0, acc_sh.at[dst0], add=True)
    plsc.subcore_barrier()

    @pl.when(s == 0)
    def _flush():
        pltpu.sync_copy(acc_sh, acc_out.at[c])


_sc_agg = pl.kernel(
    _sc_agg_body,
    out_type=jax.ShapeDtypeStruct((NC, N, D), jnp.float32),
    mesh=_MESH,
    scratch_types=[
        pltpu.VMEM_SHARED((N, D), jnp.float32),
        pltpu.VMEM((K,), jnp.int32),
        pltpu.VMEM((K,), jnp.int32),
        pltpu.VMEM((K, D), jnp.float32),
        pltpu.SemaphoreType.DMA,
        pltpu.VMEM((K,), jnp.int32),
        pltpu.VMEM((K,), jnp.int32),
        pltpu.VMEM((K, D), jnp.float32),
        pltpu.SemaphoreType.DMA,
        pltpu.VMEM((ZR, D), jnp.float32),
    ],
)


def _sc_deg_body(dst_hbm, deg_out, deg_sh, dst_v, ones_v, zrows, sem):
    del sem
    c = lax.axis_index("c")
    s = lax.axis_index("s")
    wid = c * NS + s

    _fill_rows(zrows, ZR, 0.0)
    _fill_rows(ones_v, K, 1.0)
    _zero_shared(zrows, deg_sh, s)
    plsc.subcore_barrier()

    def chunk(i, _):
        base = wid * EPW + i * K
        pltpu.sync_copy(dst_hbm.at[pl.ds(base, K)], dst_v)
        pltpu.sync_copy(ones_v, deg_sh.at[dst_v], add=True)
        return 0
    lax.fori_loop(0, NCH, chunk, 0)
    plsc.subcore_barrier()

    @pl.when(s == 0)
    def _flush():
        pltpu.sync_copy(deg_sh, deg_out.at[c])


_sc_deg = pl.kernel(
    _sc_deg_body,
    out_type=jax.ShapeDtypeStruct((NC, N, D), jnp.float32),
    mesh=_MESH,
    scratch_types=[
        pltpu.VMEM_SHARED((N, D), jnp.float32),
        pltpu.VMEM((K,), jnp.int32),
        pltpu.VMEM((K, D), jnp.float32),
        pltpu.VMEM((ZR, D), jnp.float32),
        pltpu.SemaphoreType.DMA,
    ],
)


def _dense1_body(x_ref, acc_ref, deg_ref, Wl_ref, bl_ref, Wr_ref,
                 gamma_ref, beta_ref, h_ref):
    a = acc_ref[0] + acc_ref[1]
    cnt = deg_ref[0][:, 0:1] + deg_ref[1][:, 0:1]
    mean = a / jnp.maximum(cnt, 1.0)
    out = (jnp.dot(mean, Wl_ref[...], preferred_element_type=jnp.float32)
           + bl_ref[...][None, :]
           + jnp.dot(x_ref[...], Wr_ref[...], preferred_element_type=jnp.float32))
    nrm = jnp.sqrt(jnp.sum(out * out, axis=1, keepdims=True))
    out = out / jnp.maximum(nrm, 1e-12)
    h = jnp.maximum(out, 0.0)
    mu = jnp.mean(h, axis=0, keepdims=True)
    var = jnp.mean((h - mu) * (h - mu), axis=0, keepdims=True)
    h_ref[...] = (gamma_ref[...][None, :] * (h - mu) * lax.rsqrt(var + 1e-5)
                  + beta_ref[...][None, :])


def _dense2_body(h_ref, acc_ref, deg_ref, Wl_ref, bl_ref, Wr_ref, o_ref):
    a = acc_ref[0] + acc_ref[1]
    cnt = deg_ref[0][:, 0:1] + deg_ref[1][:, 0:1]
    mean = a / jnp.maximum(cnt, 1.0)
    out = (jnp.dot(mean, Wl_ref[...], preferred_element_type=jnp.float32)
           + bl_ref[...][None, :]
           + jnp.dot(h_ref[...], Wr_ref[...], preferred_element_type=jnp.float32))
    nrm = jnp.sqrt(jnp.sum(out * out, axis=1, keepdims=True))
    o_ref[...] = out / jnp.maximum(nrm, 1e-12)


_dense1 = pl.pallas_call(
    _dense1_body, out_shape=jax.ShapeDtypeStruct((N, D), jnp.float32))
_dense2 = pl.pallas_call(
    _dense2_body, out_shape=jax.ShapeDtypeStruct((N, D), jnp.float32))


@jax.jit
def kernel(x, edge_index, W1l, b1l, W1r, W2l, b2l, W2r, gamma, beta):
    src = edge_index[0]
    dst = edge_index[1]
    deg = _sc_deg(dst)
    acc1 = _sc_agg(x, src, dst)
    h = _dense1(x, acc1, deg, W1l, b1l, W1r, gamma, beta)
    acc2 = _sc_agg(h, src, dst)
    return _dense2(h, acc2, deg, W2l, b2l, W2r)


# speedup vs baseline: 7.0610x; 1.1379x over previous
import jax
import jax.numpy as jnp
from jax import lax
from jax.experimental import pallas as pl
from jax.experimental.pallas import tpu as pltpu
from jax.experimental.pallas import tpu_sc as plsc

N = 10000
E = 320000
D = 128

NC = 2
NS = 16
NW = NC * NS
EPW = E // NW
K = 80
NCH = EPW // K
RPS = N // NS
ZR = 25

_MESH = plsc.VectorSubcoreMesh(core_axis_name="c", subcore_axis_name="s")


def _fill_rows(ref, nrows, value):
    def fill(i, _):
        for j in range(D // 16):
            ref[i, pl.ds(j * 16, 16)] = jnp.full((16,), value, jnp.float32)
        return 0
    lax.fori_loop(0, nrows, fill, 0)


def _zero_shared(zrows, sh, s):
    for r in range(RPS // ZR):
        base = s * RPS + r * ZR
        pltpu.sync_copy(zrows, sh.at[pl.ds(base, ZR)])


def _sc_agg_body(x_hbm, src_hbm, dst_hbm, acc_out, acc_sh,
                 src0, dst0, rows0, sem0, src1, dst1, rows1, sem1, zrows):
    c = lax.axis_index("c")
    s = lax.axis_index("s")
    wid = c * NS + s

    _fill_rows(zrows, ZR, 0.0)
    _zero_shared(zrows, acc_sh, s)
    plsc.subcore_barrier()

    ebase = wid * EPW

    def _load_start(i, src_v, dst_v, rows_v, sem):
        pltpu.sync_copy(src_hbm.at[pl.ds(ebase + i * K, K)], src_v)
        pltpu.sync_copy(dst_hbm.at[pl.ds(ebase + i * K, K)], dst_v)
        return pltpu.async_copy(x_hbm.at[src_v], rows_v, sem)

    def _drain(src_v, dst_v, rows_v, sem):
        pltpu.make_async_copy(x_hbm.at[src_v], rows_v, sem).wait()
        pltpu.sync_copy(rows_v, acc_sh.at[dst_v], add=True)

    _load_start(0, src0, dst0, rows0, sem0)

    def pipe(t, _):
        _load_start(2 * t + 1, src1, dst1, rows1, sem1)
        _drain(src0, dst0, rows0, sem0)
        _load_start(2 * t + 2, src0, dst0, rows0, sem0)
        _drain(src1, dst1, rows1, sem1)
        return 0
    lax.fori_loop(0, (NCH - 1) // 2, pipe, 0)
    _drain(src0, dst0, rows0, sem0)
    plsc.subcore_barrier()

    @pl.when(s == 0)
    def _flush():
        pltpu.sync_copy(acc_sh, acc_out.at[c])


_sc_agg = pl.kernel(
    _sc_agg_body,
    out_type=jax.ShapeDtypeStruct((NC, N, D), jnp.float32),
    mesh=_MESH,
    scratch_types=[
        pltpu.VMEM_SHARED((N, D), jnp.float32),
        pltpu.VMEM((K,), jnp.int32),
        pltpu.VMEM((K,), jnp.int32),
        pltpu.VMEM((K, D), jnp.float32),
        pltpu.SemaphoreType.DMA,
        pltpu.VMEM((K,), jnp.int32),
        pltpu.VMEM((K,), jnp.int32),
        pltpu.VMEM((K, D), jnp.float32),
        pltpu.SemaphoreType.DMA,
        pltpu.VMEM((ZR, D), jnp.float32),
    ],
)


def _sc_deg_body(dst_hbm, deg_out, deg_sh, dst_v, ones_v, zrows, sem):
    del sem
    c = lax.axis_index("c")
    s = lax.axis_index("s")
    wid = c * NS + s

    _fill_rows(zrows, ZR, 0.0)
    _fill_rows(ones_v, K, 1.0)
    _zero_shared(zrows, deg_sh, s)
    plsc.subcore_barrier()

    def chunk(i, _):
        base = wid * EPW + i * K
        pltpu.sync_copy(dst_hbm.at[pl.ds(base, K)], dst_v)
        pltpu.sync_copy(ones_v, deg_sh.at[dst_v], add=True)
        return 0
    lax.fori_loop(0, NCH, chunk, 0)
    plsc.subcore_barrier()

    @pl.when(s == 0)
    def _flush():
        pltpu.sync_copy(deg_sh, deg_out.at[c])


_sc_deg = pl.kernel(
    _sc_deg_body,
    out_type=jax.ShapeDtypeStruct((NC, N, D), jnp.float32),
    mesh=_MESH,
    scratch_types=[
        pltpu.VMEM_SHARED((N, D), jnp.float32),
        pltpu.VMEM((K,), jnp.int32),
        pltpu.VMEM((K, D), jnp.float32),
        pltpu.VMEM((ZR, D), jnp.float32),
        pltpu.SemaphoreType.DMA,
    ],
)


def _dense1_body(x_ref, acc_ref, deg_ref, Wl_ref, bl_ref, Wr_ref,
                 gamma_ref, beta_ref, h_ref):
    a = acc_ref[0] + acc_ref[1]
    cnt = deg_ref[0][:, 0:1] + deg_ref[1][:, 0:1]
    mean = a / jnp.maximum(cnt, 1.0)
    out = (jnp.dot(mean, Wl_ref[...], preferred_element_type=jnp.float32)
           + bl_ref[...][None, :]
           + jnp.dot(x_ref[...], Wr_ref[...], preferred_element_type=jnp.float32))
    nrm = jnp.sqrt(jnp.sum(out * out, axis=1, keepdims=True))
    out = out / jnp.maximum(nrm, 1e-12)
    h = jnp.maximum(out, 0.0)
    mu = jnp.mean(h, axis=0, keepdims=True)
    var = jnp.mean((h - mu) * (h - mu), axis=0, keepdims=True)
    h_ref[...] = (gamma_ref[...][None, :] * (h - mu) * lax.rsqrt(var + 1e-5)
                  + beta_ref[...][None, :])


def _dense2_body(h_ref, acc_ref, deg_ref, Wl_ref, bl_ref, Wr_ref, o_ref):
    a = acc_ref[0] + acc_ref[1]
    cnt = deg_ref[0][:, 0:1] + deg_ref[1][:, 0:1]
    mean = a / jnp.maximum(cnt, 1.0)
    out = (jnp.dot(mean, Wl_ref[...], preferred_element_type=jnp.float32)
           + bl_ref[...][None, :]
           + jnp.dot(h_ref[...], Wr_ref[...], preferred_element_type=jnp.float32))
    nrm = jnp.sqrt(jnp.sum(out * out, axis=1, keepdims=True))
    o_ref[...] = out / jnp.maximum(nrm, 1e-12)


_dense1 = pl.pallas_call(
    _dense1_body, out_shape=jax.ShapeDtypeStruct((N, D), jnp.float32))
_dense2 = pl.pallas_call(
    _dense2_body, out_shape=jax.ShapeDtypeStruct((N, D), jnp.float32))


@jax.jit
def kernel(x, edge_index, W1l, b1l, W1r, W2l, b2l, W2r, gamma, beta):
    src = edge_index[0]
    dst = edge_index[1]
    deg = _sc_deg(dst)
    acc1 = _sc_agg(x, src, dst)
    h = _dense1(x, acc1, deg, W1l, b1l, W1r, gamma, beta)
    acc2 = _sc_agg(h, src, dst)
    return _dense2(h, acc2, deg, W2l, b2l, W2r)


# speedup vs baseline: 7.8030x; 1.1051x over previous
import jax
import jax.numpy as jnp
from jax import lax
from jax.experimental import pallas as pl
from jax.experimental.pallas import tpu as pltpu
from jax.experimental.pallas import tpu_sc as plsc

N = 10000
E = 320000
D = 128

NC = 2
NS = 16
NW = NC * NS
EPW = E // NW
K = 80
NCH = EPW // K
RPS = N // NS
ZR = 25

_MESH = plsc.VectorSubcoreMesh(core_axis_name="c", subcore_axis_name="s")


def _fill_rows(ref, nrows, value):
    def fill(i, _):
        for j in range(D // 16):
            ref[i, pl.ds(j * 16, 16)] = jnp.full((16,), value, jnp.float32)
        return 0
    lax.fori_loop(0, nrows, fill, 0)


def _zero_shared(zrows, sh, s):
    for r in range(RPS // ZR):
        base = s * RPS + r * ZR
        pltpu.sync_copy(zrows, sh.at[pl.ds(base, ZR)])


def _sc_agg_body(x_hbm, src_hbm, dst_hbm, acc_out, acc_sh,
                 src0, dst0, rows0, sem0, src1, dst1, rows1, sem1, zrows):
    c = lax.axis_index("c")
    s = lax.axis_index("s")
    wid = c * NS + s

    _fill_rows(zrows, ZR, 0.0)
    _zero_shared(zrows, acc_sh, s)
    plsc.subcore_barrier()

    ebase = wid * EPW

    def _load_start(i, src_v, dst_v, rows_v, sem):
        pltpu.sync_copy(src_hbm.at[pl.ds(ebase + i * K, K)], src_v)
        pltpu.sync_copy(dst_hbm.at[pl.ds(ebase + i * K, K)], dst_v)
        return pltpu.async_copy(x_hbm.at[src_v], rows_v, sem)

    def _drain(src_v, dst_v, rows_v, sem):
        pltpu.make_async_copy(x_hbm.at[src_v], rows_v, sem).wait()
        pltpu.sync_copy(rows_v, acc_sh.at[dst_v], add=True)

    _load_start(0, src0, dst0, rows0, sem0)

    def pipe(t, _):
        _load_start(2 * t + 1, src1, dst1, rows1, sem1)
        _drain(src0, dst0, rows0, sem0)
        _load_start(2 * t + 2, src0, dst0, rows0, sem0)
        _drain(src1, dst1, rows1, sem1)
        return 0
    lax.fori_loop(0, (NCH - 1) // 2, pipe, 0)
    _drain(src0, dst0, rows0, sem0)
    plsc.subcore_barrier()

    @pl.when(s == 0)
    def _flush():
        pltpu.sync_copy(acc_sh, acc_out.at[c])


_sc_agg = pl.kernel(
    _sc_agg_body,
    out_type=jax.ShapeDtypeStruct((NC, N, D), jnp.float32),
    mesh=_MESH,
    scratch_types=[
        pltpu.VMEM_SHARED((N, D), jnp.float32),
        pltpu.VMEM((K,), jnp.int32),
        pltpu.VMEM((K,), jnp.int32),
        pltpu.VMEM((K, D), jnp.float32),
        pltpu.SemaphoreType.DMA,
        pltpu.VMEM((K,), jnp.int32),
        pltpu.VMEM((K,), jnp.int32),
        pltpu.VMEM((K, D), jnp.float32),
        pltpu.SemaphoreType.DMA,
        pltpu.VMEM((ZR, D), jnp.float32),
    ],
)


def _sc_deg_body(dst_hbm, deg_out, deg_sh, dst_v, dst_w, ones_v, zrows,
                 sem_v, sem_w):
    c = lax.axis_index("c")
    s = lax.axis_index("s")
    wid = c * NS + s

    _fill_rows(zrows, ZR, 0.0)
    _fill_rows(ones_v, K, 1.0)
    _zero_shared(zrows, deg_sh, s)
    plsc.subcore_barrier()

    ebase = wid * EPW
    pltpu.async_copy(dst_hbm.at[pl.ds(ebase, K)], dst_v, sem_v).wait()

    def pipe(t, _):
        lw = pltpu.async_copy(
            dst_hbm.at[pl.ds(ebase + (2 * t + 1) * K, K)], dst_w, sem_w)
        pltpu.sync_copy(ones_v, deg_sh.at[dst_v], add=True)
        lw.wait()
        lv = pltpu.async_copy(
            dst_hbm.at[pl.ds(ebase + (2 * t + 2) * K, K)], dst_v, sem_v)
        pltpu.sync_copy(ones_v, deg_sh.at[dst_w], add=True)
        lv.wait()
        return 0
    lax.fori_loop(0, (NCH - 1) // 2, pipe, 0)
    pltpu.sync_copy(ones_v, deg_sh.at[dst_v], add=True)
    plsc.subcore_barrier()

    @pl.when(s == 0)
    def _flush():
        pltpu.sync_copy(deg_sh, deg_out.at[c])


_sc_deg = pl.kernel(
    _sc_deg_body,
    out_type=jax.ShapeDtypeStruct((NC, N, D), jnp.float32),
    mesh=_MESH,
    scratch_types=[
        pltpu.VMEM_SHARED((N, D), jnp.float32),
        pltpu.VMEM((K,), jnp.int32),
        pltpu.VMEM((K,), jnp.int32),
        pltpu.VMEM((K, D), jnp.float32),
        pltpu.VMEM((ZR, D), jnp.float32),
        pltpu.SemaphoreType.DMA,
        pltpu.SemaphoreType.DMA,
    ],
)


def _dense1_body(x_ref, acc_ref, deg_ref, Wl_ref, bl_ref, Wr_ref,
                 gamma_ref, beta_ref, h_ref):
    a = acc_ref[0] + acc_ref[1]
    cnt = deg_ref[0][:, 0:1] + deg_ref[1][:, 0:1]
    mean = a / jnp.maximum(cnt, 1.0)
    out = (jnp.dot(mean, Wl_ref[...], preferred_element_type=jnp.float32)
           + bl_ref[...][None, :]
           + jnp.dot(x_ref[...], Wr_ref[...], preferred_element_type=jnp.float32))
    nrm = jnp.sqrt(jnp.sum(out * out, axis=1, keepdims=True))
    out = out / jnp.maximum(nrm, 1e-12)
    h = jnp.maximum(out, 0.0)
    mu = jnp.mean(h, axis=0, keepdims=True)
    var = jnp.mean((h - mu) * (h - mu), axis=0, keepdims=True)
    h_ref[...] = (gamma_ref[...][None, :] * (h - mu) * lax.rsqrt(var + 1e-5)
                  + beta_ref[...][None, :])


def _dense2_body(h_ref, acc_ref, deg_ref, Wl_ref, bl_ref, Wr_ref, o_ref):
    a = acc_ref[0] + acc_ref[1]
    cnt = deg_ref[0][:, 0:1] + deg_ref[1][:, 0:1]
    mean = a / jnp.maximum(cnt, 1.0)
    out = (jnp.dot(mean, Wl_ref[...], preferred_element_type=jnp.float32)
           + bl_ref[...][None, :]
           + jnp.dot(h_ref[...], Wr_ref[...], preferred_element_type=jnp.float32))
    nrm = jnp.sqrt(jnp.sum(out * out, axis=1, keepdims=True))
    o_ref[...] = out / jnp.maximum(nrm, 1e-12)


_dense1 = pl.pallas_call(
    _dense1_body, out_shape=jax.ShapeDtypeStruct((N, D), jnp.float32))
_dense2 = pl.pallas_call(
    _dense2_body, out_shape=jax.ShapeDtypeStruct((N, D), jnp.float32))


@jax.jit
def kernel(x, edge_index, W1l, b1l, W1r, W2l, b2l, W2r, gamma, beta):
    src = edge_index[0]
    dst = edge_index[1]
    deg = _sc_deg(dst)
    acc1 = _sc_agg(x, src, dst)
    h = _dense1(x, acc1, deg, W1l, b1l, W1r, gamma, beta)
    acc2 = _sc_agg(h, src, dst)
    return _dense2(h, acc2, deg, W2l, b2l, W2r)
